# baseline (speedup 1.0000x reference)
import jax
import jax.numpy as jnp
from jax import lax
from jax.experimental import pallas as pl
from jax.experimental.pallas import tpu as pltpu

_B, _T = 128, 50
_N_STATIC = 64
_EN, _ET, _H = 64, 128, 512
_D = _ET
_VN_PAD = 1024
_CHUNK = 8


def _rnn_step_kernel(
    nin_ref, tin_ref, embN_ref, embT_ref,
    wall_ref, ball_ref,
    h0_ref, c0_ref, dinit_ref,
    out_ref,
    h_ref, c_ref, buf_ref, acc_ref,
):
    t = pl.program_id(0)

    @pl.when(t == 0)
    def _init():
        h_ref[...] = jnp.broadcast_to(h0_ref[...], (_B, _H))
        c_ref[...] = jnp.broadcast_to(c0_ref[...], (_B, _H))

    tin = tin_ref[...]
    nin = nin_ref[...]
    iota_t = lax.broadcasted_iota(jnp.int32, (_B, _T), 1)
    step_mask = iota_t == t
    tin_col = jnp.sum(jnp.where(step_mask, tin, 0), axis=1, keepdims=True)
    nin_col = jnp.sum(jnp.where(step_mask, nin, 0), axis=1, keepdims=True)
    is_static = tin_col < _N_STATIC

    match = (tin == tin_col) & (tin >= _N_STATIC) & (tin_col >= _N_STATIC) \
        & (iota_t < t)
    prev = jnp.max(jnp.where(match, iota_t, -1), axis=1, keepdims=True)
    has_prev = prev >= 0

    acc_ref[...] = jnp.zeros((_B, 2 * _D), jnp.bfloat16)

    def _chunk(i, carry):
        base = i * _CHUNK
        blk = buf_ref[pl.ds(base, _CHUNK)]
        contrib = jnp.zeros((_B, 2 * _D), jnp.bfloat16)
        for j in range(_CHUNK):
            contrib = contrib + jnp.where(prev == base + j, blk[j],
                                          jnp.bfloat16(0.0))
        acc_ref[...] = acc_ref[...] + contrib
        return carry

    nchunks = (t + _CHUNK - 1) // _CHUNK
    lax.fori_loop(0, nchunks, _chunk, 0)
    upd = acc_ref[...].astype(jnp.float32)

    iota_n = lax.broadcasted_iota(jnp.int32, (_B, _VN_PAD), 1)
    onehot_n = (iota_n == nin_col).astype(jnp.float32)
    n_emb = jnp.dot(onehot_n, embN_ref[...], preferred_element_type=jnp.float32)

    t_clip = jnp.clip(tin_col, 0, _N_STATIC - 1)
    iota_s = lax.broadcasted_iota(jnp.int32, (_B, _N_STATIC), 1)
    onehot_s = (iota_s == t_clip).astype(jnp.float32)
    stat_emb = jnp.dot(onehot_s, embT_ref[...], preferred_element_type=jnp.float32)

    dinit = jnp.broadcast_to(dinit_ref[...], (_B, 2 * _D))
    h_dynamic = jnp.where(has_prev, upd[:, :_D], dinit[:, :_D])
    c_dynamic = jnp.where(has_prev, upd[:, _D:], dinit[:, _D:])
    h_tensor = jnp.where(is_static, stat_emb, h_dynamic)

    h = h_ref[...]
    c = c_ref[...]

    x2 = jnp.concatenate([h_tensor, h_dynamic, n_emb], axis=1)
    gall = (jnp.dot(h, wall_ref[:_H], preferred_element_type=jnp.float32)
            + jnp.dot(x2, wall_ref[_H:], preferred_element_type=jnp.float32)
            + ball_ref[...])
    g = gall[:, :4 * _H]
    gd = gall[:, 4 * _H:]
    gi = jax.nn.sigmoid(g[:, 0 * _H:1 * _H])
    gf = jax.nn.sigmoid(g[:, 1 * _H:2 * _H])
    gg = jnp.tanh(g[:, 2 * _H:3 * _H])
    go = jax.nn.sigmoid(g[:, 3 * _H:4 * _H])
    c_new = gf * c + gi * gg
    h_new = go * jnp.tanh(c_new)

    di = jax.nn.sigmoid(gd[:, 0 * _D:1 * _D])
    df = jax.nn.sigmoid(gd[:, 1 * _D:2 * _D])
    dg = jnp.tanh(gd[:, 2 * _D:3 * _D])
    do = jax.nn.sigmoid(gd[:, 3 * _D:4 * _D])
    c_dyn = df * c_dynamic + di * dg
    h_dyn = do * jnp.tanh(c_dyn)

    buf_ref[pl.ds(t, 1)] = jnp.concatenate(
        [h_dyn, c_dyn], axis=1).astype(jnp.bfloat16)[None]
    h_ref[...] = h_new
    c_ref[...] = c_new
    out_ref[...] = h_new[None, :, :]


@jax.jit
def kernel(n_input_all, t_input_all, embN, embT, Wih, Whh, bih, bhh,
           Wih_d, Whh_d, bih_d, bhh_d, hid_init, cell_init, dyn_init_h, dyn_init_c):
    nin = n_input_all.astype(jnp.int32)
    tin = t_input_all.astype(jnp.int32)
    embN_pad = jnp.zeros((_VN_PAD, _EN), jnp.float32).at[:embN.shape[0]].set(embN)

    K = _H + _D + _D + _EN
    wall = jnp.zeros((K, 4 * _H + 4 * _D), jnp.float32)
    wall = wall.at[:_H, :4 * _H].set(Whh.T)
    wall = wall.at[:_H, 4 * _H:].set(Wih_d[:, _EN:].T)
    wall = wall.at[_H:_H + _D, :4 * _H].set(Wih[:, _EN:].T)
    wall = wall.at[_H + _D:_H + 2 * _D, 4 * _H:].set(Whh_d.T)
    wall = wall.at[_H + 2 * _D:, :4 * _H].set(Wih[:, :_EN].T)
    wall = wall.at[_H + 2 * _D:, 4 * _H:].set(Wih_d[:, :_EN].T)
    ball = jnp.concatenate([bih + bhh, bih_d + bhh_d])[None, :]
    dinit = jnp.concatenate([dyn_init_h, dyn_init_c], axis=1)

    full = lambda shape: pl.BlockSpec(shape, lambda t: tuple(0 for _ in shape))
    grid_spec = pltpu.PrefetchScalarGridSpec(
        num_scalar_prefetch=0,
        grid=(_T,),
        in_specs=[
            full((_B, _T)), full((_B, _T)),
            full((_VN_PAD, _EN)), full((_N_STATIC, _ET)),
            full((_H + 2 * _D + _EN, 4 * _H + 4 * _D)), full((1, 4 * _H + 4 * _D)),
            full((1, _H)), full((1, _H)), full((1, 2 * _D)),
        ],
        out_specs=pl.BlockSpec((1, _B, _H), lambda t: (t, 0, 0)),
        scratch_shapes=[
            pltpu.VMEM((_B, _H), jnp.float32),
            pltpu.VMEM((_B, _H), jnp.float32),
            pltpu.VMEM((_T + _CHUNK, _B, 2 * _D), jnp.bfloat16),
            pltpu.VMEM((_B, 2 * _D), jnp.bfloat16),
        ],
    )
    out = pl.pallas_call(
        _rnn_step_kernel,
        grid_spec=grid_spec,
        out_shape=jax.ShapeDtypeStruct((_T, _B, _H), jnp.float32),
    )(nin, tin, embN_pad, embT, wall, ball,
      hid_init[None, :], cell_init[None, :], dinit)
    return out.transpose(1, 0, 2)


# speedup vs baseline: 1.1427x; 1.1427x over previous
import jax
import jax.numpy as jnp
from jax import lax
from jax.experimental import pallas as pl
from jax.experimental.pallas import tpu as pltpu

_B, _T = 128, 50
_N_STATIC = 64
_EN, _ET, _H = 64, 128, 512
_D = _ET
_VN_PAD = 1024
_CHUNK = 16


def _rnn_step_kernel(
    nin_ref, tin_ref, embN_ref, embT_ref,
    wihA_ref, wihB_ref, whh_ref, bmain_ref,
    wdA_ref, wdB_ref, wdH_ref, bdyn_ref,
    h0_ref, c0_ref, dinit_ref,
    out_ref,
    h_ref, c_ref, buf_ref, acc_ref,
):
    t = pl.program_id(0)

    @pl.when(t == 0)
    def _init():
        h_ref[...] = jnp.broadcast_to(h0_ref[...], (_B, _H))
        c_ref[...] = jnp.broadcast_to(c0_ref[...], (_B, _H))

    tin = tin_ref[...]
    nin = nin_ref[...]
    iota_t = lax.broadcasted_iota(jnp.int32, (_B, _T), 1)
    step_mask = iota_t == t
    tin_col = jnp.sum(jnp.where(step_mask, tin, 0), axis=1, keepdims=True)
    nin_col = jnp.sum(jnp.where(step_mask, nin, 0), axis=1, keepdims=True)
    is_static = tin_col < _N_STATIC

    match = (tin == tin_col) & (tin >= _N_STATIC) & (tin_col >= _N_STATIC) \
        & (iota_t < t)
    prev = jnp.max(jnp.where(match, iota_t, -1), axis=1, keepdims=True)
    has_prev = prev >= 0

    acc_ref[...] = jnp.zeros((_B, 2 * _D), jnp.bfloat16)

    def _chunk(i, carry):
        base = i * _CHUNK
        blk = buf_ref[pl.ds(base, _CHUNK)]
        contrib = jnp.zeros((_B, 2 * _D), jnp.bfloat16)
        for j in range(_CHUNK):
            contrib = contrib + jnp.where(prev == base + j, blk[j],
                                          jnp.bfloat16(0.0))
        acc_ref[...] = acc_ref[...] + contrib
        return carry

    nchunks = (t + _CHUNK - 1) // _CHUNK
    lax.fori_loop(0, nchunks, _chunk, 0)
    upd = acc_ref[...].astype(jnp.float32)

    iota_n = lax.broadcasted_iota(jnp.int32, (_B, _VN_PAD), 1)
    onehot_n = (iota_n == nin_col).astype(jnp.float32)
    n_emb = jnp.dot(onehot_n, embN_ref[...], preferred_element_type=jnp.float32)

    t_clip = jnp.clip(tin_col, 0, _N_STATIC - 1)
    iota_s = lax.broadcasted_iota(jnp.int32, (_B, _N_STATIC), 1)
    onehot_s = (iota_s == t_clip).astype(jnp.float32)
    stat_emb = jnp.dot(onehot_s, embT_ref[...], preferred_element_type=jnp.float32)

    dinit = jnp.broadcast_to(dinit_ref[...], (_B, 2 * _D))
    h_dynamic = jnp.where(has_prev, upd[:, :_D], dinit[:, :_D])
    c_dynamic = jnp.where(has_prev, upd[:, _D:], dinit[:, _D:])
    h_tensor = jnp.where(is_static, stat_emb, h_dynamic)

    h = h_ref[...]
    c = c_ref[...]

    g = (jnp.dot(n_emb, wihA_ref[...], preferred_element_type=jnp.float32)
         + jnp.dot(h_tensor, wihB_ref[...], preferred_element_type=jnp.float32)
         + jnp.dot(h, whh_ref[...], preferred_element_type=jnp.float32)
         + bmain_ref[...])
    gi = jax.nn.sigmoid(g[:, 0 * _H:1 * _H])
    gf = jax.nn.sigmoid(g[:, 1 * _H:2 * _H])
    gg = jnp.tanh(g[:, 2 * _H:3 * _H])
    go = jax.nn.sigmoid(g[:, 3 * _H:4 * _H])
    c_new = gf * c + gi * gg
    h_new = go * jnp.tanh(c_new)

    gd = (jnp.dot(n_emb, wdA_ref[...], preferred_element_type=jnp.float32)
          + jnp.dot(h, wdB_ref[...], preferred_element_type=jnp.float32)
          + jnp.dot(h_dynamic, wdH_ref[...], preferred_element_type=jnp.float32)
          + bdyn_ref[...])
    di = jax.nn.sigmoid(gd[:, 0 * _D:1 * _D])
    df = jax.nn.sigmoid(gd[:, 1 * _D:2 * _D])
    dg = jnp.tanh(gd[:, 2 * _D:3 * _D])
    do = jax.nn.sigmoid(gd[:, 3 * _D:4 * _D])
    c_dyn = df * c_dynamic + di * dg
    h_dyn = do * jnp.tanh(c_dyn)

    buf_ref[pl.ds(t, 1)] = jnp.concatenate(
        [h_dyn, c_dyn], axis=1).astype(jnp.bfloat16)[None]
    h_ref[...] = h_new
    c_ref[...] = c_new
    out_ref[...] = h_new[None, :, :]


@jax.jit
def kernel(n_input_all, t_input_all, embN, embT, Wih, Whh, bih, bhh,
           Wih_d, Whh_d, bih_d, bhh_d, hid_init, cell_init, dyn_init_h, dyn_init_c):
    nin = n_input_all.astype(jnp.int32)
    tin = t_input_all.astype(jnp.int32)
    embN_pad = jnp.zeros((_VN_PAD, _EN), jnp.float32).at[:embN.shape[0]].set(embN)

    wihA = Wih[:, :_EN].T
    wihB = Wih[:, _EN:].T
    whh = Whh.T
    bmain = (bih + bhh)[None, :]
    wdA = Wih_d[:, :_EN].T
    wdB = Wih_d[:, _EN:].T
    wdH = Whh_d.T
    bdyn = (bih_d + bhh_d)[None, :]
    dinit = jnp.concatenate([dyn_init_h, dyn_init_c], axis=1)

    full = lambda shape: pl.BlockSpec(shape, lambda t: tuple(0 for _ in shape))
    grid_spec = pltpu.PrefetchScalarGridSpec(
        num_scalar_prefetch=0,
        grid=(_T,),
        in_specs=[
            full((_B, _T)), full((_B, _T)),
            full((_VN_PAD, _EN)), full((_N_STATIC, _ET)),
            full((_EN, 4 * _H)), full((_D, 4 * _H)), full((_H, 4 * _H)), full((1, 4 * _H)),
            full((_EN, 4 * _D)), full((_H, 4 * _D)), full((_D, 4 * _D)), full((1, 4 * _D)),
            full((1, _H)), full((1, _H)), full((1, 2 * _D)),
        ],
        out_specs=pl.BlockSpec((1, _B, _H), lambda t: (t, 0, 0)),
        scratch_shapes=[
            pltpu.VMEM((_B, _H), jnp.float32),
            pltpu.VMEM((_B, _H), jnp.float32),
            pltpu.VMEM((_T + _CHUNK, _B, 2 * _D), jnp.bfloat16),
            pltpu.VMEM((_B, 2 * _D), jnp.bfloat16),
        ],
    )
    out = pl.pallas_call(
        _rnn_step_kernel,
        grid_spec=grid_spec,
        out_shape=jax.ShapeDtypeStruct((_T, _B, _H), jnp.float32),
    )(nin, tin, embN_pad, embT,
      wihA, wihB, whh, bmain,
      wdA, wdB, wdH, bdyn,
      hid_init[None, :], cell_init[None, :], dinit)
    return out.transpose(1, 0, 2)


# speedup vs baseline: 1.1559x; 1.0115x over previous
import jax
import jax.numpy as jnp
from jax import lax
from jax.experimental import pallas as pl
from jax.experimental.pallas import tpu as pltpu

_B, _T = 128, 50
_N_STATIC = 64
_EN, _ET, _H = 64, 128, 512
_D = _ET
_VN_PAD = 1024
_CHUNK = 8


def _rnn_step_kernel(
    nin_ref, tin_ref, embN_ref, embT_ref,
    wihA_ref, wihB_ref, whh_ref, bmain_ref,
    wdA_ref, wdB_ref, wdH_ref, bdyn_ref,
    h0_ref, c0_ref, dinit_ref,
    out_ref,
    h_ref, c_ref, buf_ref, acc_ref,
):
    t = pl.program_id(0)

    @pl.when(t == 0)
    def _init():
        h_ref[...] = jnp.broadcast_to(h0_ref[...], (_B, _H))
        c_ref[...] = jnp.broadcast_to(c0_ref[...], (_B, _H))

    tin = tin_ref[...]
    nin = nin_ref[...]
    iota_t = lax.broadcasted_iota(jnp.int32, (_B, _T), 1)
    step_mask = iota_t == t
    tin_col = jnp.sum(jnp.where(step_mask, tin, 0), axis=1, keepdims=True)
    nin_col = jnp.sum(jnp.where(step_mask, nin, 0), axis=1, keepdims=True)
    is_static = tin_col < _N_STATIC

    match = (tin == tin_col) & (tin >= _N_STATIC) & (tin_col >= _N_STATIC) \
        & (iota_t < t)
    prev = jnp.max(jnp.where(match, iota_t, -1), axis=1, keepdims=True)
    has_prev = prev >= 0

    acc_ref[...] = jnp.zeros((_B, 2 * _D), jnp.bfloat16)

    def _chunk(i, carry):
        base = i * _CHUNK
        blk = buf_ref[pl.ds(base, _CHUNK)]
        contrib = jnp.zeros((_B, 2 * _D), jnp.bfloat16)
        for j in range(_CHUNK):
            contrib = contrib + jnp.where(prev == base + j, blk[j],
                                          jnp.bfloat16(0.0))
        acc_ref[...] = acc_ref[...] + contrib
        return carry

    nchunks = (t + _CHUNK - 1) // _CHUNK
    lax.fori_loop(0, nchunks, _chunk, 0)
    upd = acc_ref[...].astype(jnp.float32)

    iota_n = lax.broadcasted_iota(jnp.int32, (_B, _VN_PAD), 1)
    onehot_n = (iota_n == nin_col).astype(jnp.float32)
    n_emb = jnp.dot(onehot_n, embN_ref[...], preferred_element_type=jnp.float32)

    t_clip = jnp.clip(tin_col, 0, _N_STATIC - 1)
    iota_s = lax.broadcasted_iota(jnp.int32, (_B, _N_STATIC), 1)
    onehot_s = (iota_s == t_clip).astype(jnp.float32)
    stat_emb = jnp.dot(onehot_s, embT_ref[...], preferred_element_type=jnp.float32)

    dinit = jnp.broadcast_to(dinit_ref[...], (_B, 2 * _D))
    h_dynamic = jnp.where(has_prev, upd[:, :_D], dinit[:, :_D])
    c_dynamic = jnp.where(has_prev, upd[:, _D:], dinit[:, _D:])
    h_tensor = jnp.where(is_static, stat_emb, h_dynamic)

    h = h_ref[...]
    c = c_ref[...]

    g = (jnp.dot(n_emb, wihA_ref[...], preferred_element_type=jnp.float32)
         + jnp.dot(h_tensor, wihB_ref[...], preferred_element_type=jnp.float32)
         + jnp.dot(h, whh_ref[...], preferred_element_type=jnp.float32)
         + bmain_ref[...])
    gi = jax.nn.sigmoid(g[:, 0 * _H:1 * _H])
    gf = jax.nn.sigmoid(g[:, 1 * _H:2 * _H])
    gg = jnp.tanh(g[:, 2 * _H:3 * _H])
    go = jax.nn.sigmoid(g[:, 3 * _H:4 * _H])
    c_new = gf * c + gi * gg
    h_new = go * jnp.tanh(c_new)

    gd = (jnp.dot(n_emb, wdA_ref[...], preferred_element_type=jnp.float32)
          + jnp.dot(h, wdB_ref[...], preferred_element_type=jnp.float32)
          + jnp.dot(h_dynamic, wdH_ref[...], preferred_element_type=jnp.float32)
          + bdyn_ref[...])
    di = jax.nn.sigmoid(gd[:, 0 * _D:1 * _D])
    df = jax.nn.sigmoid(gd[:, 1 * _D:2 * _D])
    dg = jnp.tanh(gd[:, 2 * _D:3 * _D])
    do = jax.nn.sigmoid(gd[:, 3 * _D:4 * _D])
    c_dyn = df * c_dynamic + di * dg
    h_dyn = do * jnp.tanh(c_dyn)

    buf_ref[pl.ds(t, 1)] = jnp.concatenate(
        [h_dyn, c_dyn], axis=1).astype(jnp.bfloat16)[None]
    h_ref[...] = h_new
    c_ref[...] = c_new
    out_ref[...] = h_new[None, :, :]


@jax.jit
def kernel(n_input_all, t_input_all, embN, embT, Wih, Whh, bih, bhh,
           Wih_d, Whh_d, bih_d, bhh_d, hid_init, cell_init, dyn_init_h, dyn_init_c):
    nin = n_input_all.astype(jnp.int32)
    tin = t_input_all.astype(jnp.int32)
    embN_pad = jnp.zeros((_VN_PAD, _EN), jnp.float32).at[:embN.shape[0]].set(embN)

    wihA = Wih[:, :_EN].T
    wihB = Wih[:, _EN:].T
    whh = Whh.T
    bmain = (bih + bhh)[None, :]
    wdA = Wih_d[:, :_EN].T
    wdB = Wih_d[:, _EN:].T
    wdH = Whh_d.T
    bdyn = (bih_d + bhh_d)[None, :]
    dinit = jnp.concatenate([dyn_init_h, dyn_init_c], axis=1)

    full = lambda shape: pl.BlockSpec(shape, lambda t: tuple(0 for _ in shape))
    grid_spec = pltpu.PrefetchScalarGridSpec(
        num_scalar_prefetch=0,
        grid=(_T,),
        in_specs=[
            full((_B, _T)), full((_B, _T)),
            full((_VN_PAD, _EN)), full((_N_STATIC, _ET)),
            full((_EN, 4 * _H)), full((_D, 4 * _H)), full((_H, 4 * _H)), full((1, 4 * _H)),
            full((_EN, 4 * _D)), full((_H, 4 * _D)), full((_D, 4 * _D)), full((1, 4 * _D)),
            full((1, _H)), full((1, _H)), full((1, 2 * _D)),
        ],
        out_specs=pl.BlockSpec((1, _B, _H), lambda t: (t, 0, 0)),
        scratch_shapes=[
            pltpu.VMEM((_B, _H), jnp.float32),
            pltpu.VMEM((_B, _H), jnp.float32),
            pltpu.VMEM((_T + _CHUNK, _B, 2 * _D), jnp.bfloat16),
            pltpu.VMEM((_B, 2 * _D), jnp.bfloat16),
        ],
    )
    out = pl.pallas_call(
        _rnn_step_kernel,
        grid_spec=grid_spec,
        out_shape=jax.ShapeDtypeStruct((_T, _B, _H), jnp.float32),
    )(nin, tin, embN_pad, embT,
      wihA, wihB, whh, bmain,
      wdA, wdB, wdH, bdyn,
      hid_init[None, :], cell_init[None, :], dinit)
    return out.transpose(1, 0, 2)
